# Initial kernel scaffold; baseline (speedup 1.0000x reference)
import jax
import jax.numpy as jnp
from jax.experimental import pallas as pl


def kernel(pts, edges, W1, b1, W2, b2, W3, b3):
    raise NotImplementedError("write your pallas kernel here")


# speedup vs baseline: 4.4483x; 4.4483x over previous
import functools

import jax
import jax.numpy as jnp
from jax import lax
from jax.experimental import pallas as pl
from jax.experimental.pallas import tpu as pltpu
from jax.experimental.pallas import tpu_sc as plsc

N = 8192
D = 8
K = 32
E = 100000
H1 = 512
H2 = 512

R = 128
NW = 32
EPAD = 100352
EPW = EPAD // NW


def _leaky(x):
    return jnp.where(x >= 0, x, 0.01 * x)


def _vertex_body(pts_blk_ref, pts_all_ref, w1_ref, b1_ref, w2_ref, b2_ref,
                 w3_ref, b3_ref, out_ref):
    pts_blk = pts_blk_ref[:]
    pts_all = pts_all_ref[:]
    g = lax.dot_general(pts_blk, pts_all, (((1,), (1,)), ((), ())),
                        preferred_element_type=jnp.float32)
    sq_blk = jnp.sum(pts_blk * pts_blk, axis=1, keepdims=True)
    sq_all = jnp.sum(pts_all * pts_all, axis=1, keepdims=True)
    d2 = sq_blk + sq_all.T - 2.0 * g
    d2 = jnp.maximum(d2, 1e-12)
    ki = lax.bitcast_convert_type(d2, jnp.int32)
    cols = lax.broadcasted_iota(jnp.int32, (R, N), 1)
    vals = []
    for k in range(K + 1):
        m = jnp.min(ki, axis=1, keepdims=True)
        if k > 0:
            vals.append(m)
        if k < K:
            first = jnp.min(jnp.where(ki == m, cols, N), axis=1,
                            keepdims=True)
            ki = jnp.where(cols == first, jnp.int32(0x7FFFFFFF), ki)
    knn_i = jnp.concatenate(vals, axis=1)
    knn = jnp.sqrt(lax.bitcast_convert_type(knn_i, jnp.float32))
    h = _leaky(jnp.dot(knn, w1_ref[:], preferred_element_type=jnp.float32)
               + b1_ref[:])
    h = _leaky(jnp.dot(h, w2_ref[:], preferred_element_type=jnp.float32)
               + b2_ref[:])
    f = jnp.sum(h * w3_ref[:], axis=1) + b3_ref[0, 0]
    out_ref[0, 0, :] = f


def _vertex_filts(pts, W1, b1, W2, b2, w3row, b3):
    grid = N // R
    out = pl.pallas_call(
        _vertex_body,
        grid=(grid,),
        in_specs=[
            pl.BlockSpec((R, D), lambda i: (i, 0)),
            pl.BlockSpec((N, D), lambda i: (0, 0)),
            pl.BlockSpec((K, H1), lambda i: (0, 0)),
            pl.BlockSpec((1, H1), lambda i: (0, 0)),
            pl.BlockSpec((H1, H2), lambda i: (0, 0)),
            pl.BlockSpec((1, H2), lambda i: (0, 0)),
            pl.BlockSpec((1, H2), lambda i: (0, 0)),
            pl.BlockSpec((1, 1), lambda i: (0, 0)),
        ],
        out_specs=pl.BlockSpec((1, 1, R), lambda i: (i, 0, 0)),
        out_shape=jax.ShapeDtypeStruct((grid, 1, R), jnp.float32),
    )(pts, pts, W1, b1.reshape(1, H1), W2, b2.reshape(1, H2), w3row, b3)
    return out.reshape(N)


def _sqrt_sc(x):
    xi = lax.bitcast_convert_type(x, jnp.int32)
    yi = jnp.int32(0x1FBD1DF5) + (xi >> 1)
    y = lax.bitcast_convert_type(yi, jnp.float32)
    for _ in range(3):
        y = 0.5 * (y + x / y)
    return y


def _edge_body(ptsf_hbm, f_hbm, u_hbm, v_hbm, out_hbm,
               u_v, v_v, iu_v, iv_v, pu_v, pv_v, acc_v, fu_v, fv_v, out_v,
               sem):
    wid = lax.axis_index("s") * 2 + lax.axis_index("c")
    base = wid * EPW
    steps = EPW // 16
    pltpu.sync_copy(u_hbm.at[pl.ds(base, EPW)], u_v)
    pltpu.sync_copy(v_hbm.at[pl.ds(base, EPW)], v_v)
    cu = pltpu.async_copy(f_hbm.at[u_v], fu_v, sem)
    cv = pltpu.async_copy(f_hbm.at[v_v], fv_v, sem)

    def init(i, c):
        sl = pl.ds(i * 16, 16)
        acc_v[sl] = jnp.zeros((16,), jnp.float32)
        iu_v[sl] = u_v[sl] * D
        iv_v[sl] = v_v[sl] * D
        return c

    lax.fori_loop(0, steps, init, 0)
    cu.wait()
    cv.wait()
    for d in range(D):
        gu = pltpu.async_copy(ptsf_hbm.at[iu_v], pu_v, sem)
        gv = pltpu.async_copy(ptsf_hbm.at[iv_v], pv_v, sem)
        gu.wait()
        gv.wait()

        def accd(i, c):
            sl = pl.ds(i * 16, 16)
            df = pu_v[sl] - pv_v[sl]
            acc_v[sl] = acc_v[sl] + df * df
            iu_v[sl] = iu_v[sl] + 1
            iv_v[sl] = iv_v[sl] + 1
            return c

        lax.fori_loop(0, steps, accd, 0)

    def fin(i, c):
        sl = pl.ds(i * 16, 16)
        fm = jnp.maximum(fu_v[sl], fv_v[sl])
        out_v[sl] = _sqrt_sc(jnp.maximum(acc_v[sl], 1e-12)) + fm
        return c

    lax.fori_loop(0, steps, fin, 0)
    pltpu.sync_copy(out_v, out_hbm.at[pl.ds(base, EPW)])


def _edge_filts(ptsf, vfilts, u_pad, v_pad):
    mesh = plsc.VectorSubcoreMesh(core_axis_name="c", subcore_axis_name="s")
    call = pl.kernel(
        _edge_body,
        mesh=mesh,
        out_type=jax.ShapeDtypeStruct((EPAD,), jnp.float32),
        scratch_types=[
            pltpu.VMEM((EPW,), jnp.int32),
            pltpu.VMEM((EPW,), jnp.int32),
            pltpu.VMEM((EPW,), jnp.int32),
            pltpu.VMEM((EPW,), jnp.int32),
            pltpu.VMEM((EPW,), jnp.float32),
            pltpu.VMEM((EPW,), jnp.float32),
            pltpu.VMEM((EPW,), jnp.float32),
            pltpu.VMEM((EPW,), jnp.float32),
            pltpu.VMEM((EPW,), jnp.float32),
            pltpu.VMEM((EPW,), jnp.float32),
            pltpu.SemaphoreType.DMA,
        ],
    )
    return call(ptsf, vfilts, u_pad, v_pad)


def kernel(pts, edges, W1, b1, W2, b2, W3, b3):
    edges = edges.astype(jnp.int32)
    vfilts = _vertex_filts(pts, W1, b1, W2, b2, W3.reshape(1, H2),
                           b3.reshape(1, 1))
    pad = jnp.zeros((EPAD - E,), jnp.int32)
    u_pad = jnp.concatenate([edges[:, 0], pad])
    v_pad = jnp.concatenate([edges[:, 1], pad])
    efilts = _edge_filts(pts.reshape(-1), vfilts, u_pad, v_pad)
    return vfilts, efilts[:E]
